# Initial kernel scaffold; baseline (speedup 1.0000x reference)
import jax
import jax.numpy as jnp
from jax.experimental import pallas as pl


def kernel(inputs, table, s_cls_token):
    raise NotImplementedError("write your pallas kernel here")


# speedup vs baseline: 3.5296x; 3.5296x over previous
import functools

import jax
import jax.numpy as jnp
from jax import lax
from jax.experimental import pallas as pl
from jax.experimental.pallas import tpu as pltpu
from jax.experimental.pallas import tpu_sc as plsc

N_CHAR = 1000
HIDDEN = 64
BATCH = 4096
SEQ = 200
CHUNK = 100


def kernel(inputs, table, s_cls_token):
    info = plsc.get_sparse_core_info()
    nc, ns = info.num_cores, info.num_subcores
    nw = nc * ns
    b_per_w = BATCH // nw

    idx3 = inputs.reshape(BATCH, SEQ // CHUNK, CHUNK).astype(jnp.int32)
    cls_row = s_cls_token.reshape(1, HIDDEN).astype(jnp.float32)

    mesh = plsc.VectorSubcoreMesh(core_axis_name="c", subcore_axis_name="s")

    @functools.partial(
        pl.kernel,
        mesh=mesh,
        out_type=jax.ShapeDtypeStruct((BATCH, SEQ + 1, HIDDEN), jnp.float32),
        scratch_types=[
            pltpu.VMEM((SEQ // CHUNK, CHUNK), jnp.int32),
            pltpu.VMEM((SEQ + 1, HIDDEN), jnp.float32),
            pltpu.SemaphoreType.DMA,
        ],
        compiler_params=pltpu.CompilerParams(use_tc_tiling_on_sc=False),
    )
    def emb_kernel(idx_hbm, table_hbm, cls_hbm, out_hbm, idx_v, buf_v, sem):
        wid = lax.axis_index("s") * nc + lax.axis_index("c")
        base = wid * b_per_w
        pltpu.sync_copy(cls_hbm, buf_v.at[pl.ds(0, 1)])

        def body(i, carry):
            b = base + i
            pltpu.sync_copy(idx_hbm.at[b], idx_v)
            cp0 = pltpu.async_copy(
                table_hbm.at[idx_v.at[0]], buf_v.at[pl.ds(1, CHUNK)], sem
            )
            cp1 = pltpu.async_copy(
                table_hbm.at[idx_v.at[1]], buf_v.at[pl.ds(1 + CHUNK, CHUNK)], sem
            )
            cp0.wait()
            cp1.wait()
            pltpu.sync_copy(buf_v, out_hbm.at[b])
            return carry

        lax.fori_loop(0, b_per_w, body, 0)

    return emb_kernel(idx3, table, cls_row)


# speedup vs baseline: 3.6822x; 1.0432x over previous
import functools

import jax
import jax.numpy as jnp
from jax import lax
from jax.experimental import pallas as pl
from jax.experimental.pallas import tpu as pltpu
from jax.experimental.pallas import tpu_sc as plsc

N_CHAR = 1000
HIDDEN = 64
BATCH = 4096
SEQ = 200
CHUNK = 100
NBUF = 4


def kernel(inputs, table, s_cls_token):
    info = plsc.get_sparse_core_info()
    nc, ns = info.num_cores, info.num_subcores
    nw = nc * ns
    b_per_w = BATCH // nw

    idx3 = inputs.reshape(BATCH, SEQ // CHUNK, CHUNK).astype(jnp.int32)
    cls_row = s_cls_token.reshape(1, HIDDEN).astype(jnp.float32)

    mesh = plsc.VectorSubcoreMesh(core_axis_name="c", subcore_axis_name="s")

    @functools.partial(
        pl.kernel,
        mesh=mesh,
        out_type=jax.ShapeDtypeStruct((BATCH, SEQ + 1, HIDDEN), jnp.float32),
        scratch_types=[
            pltpu.VMEM((BATCH // 32, SEQ // CHUNK, CHUNK), jnp.int32),
            pltpu.VMEM((NBUF, SEQ + 1, HIDDEN), jnp.float32),
            pltpu.SemaphoreType.DMA((NBUF,)),
            pltpu.SemaphoreType.DMA((NBUF,)),
        ],
        compiler_params=pltpu.CompilerParams(use_tc_tiling_on_sc=False),
    )
    def emb_kernel(idx_hbm, table_hbm, cls_hbm, out_hbm, idx_v, bufs, gsem, wsem):
        wid = lax.axis_index("s") * nc + lax.axis_index("c")
        base = wid * b_per_w

        def start_gather(i, b):
            pltpu.async_copy(
                table_hbm.at[idx_v.at[i, 0]], bufs.at[b, pl.ds(1, CHUNK)],
                gsem.at[b])
            pltpu.async_copy(
                table_hbm.at[idx_v.at[i, 1]], bufs.at[b, pl.ds(1 + CHUNK, CHUNK)],
                gsem.at[b])

        def wait_gather(b):
            pltpu.make_async_copy(
                table_hbm.at[pl.ds(0, SEQ)], bufs.at[b, pl.ds(1, SEQ)],
                gsem.at[b]).wait()

        def start_write(i, b):
            pltpu.async_copy(bufs.at[b], out_hbm.at[base + i], wsem.at[b])

        def wait_write(i, b):
            pltpu.make_async_copy(bufs.at[b], out_hbm.at[base + i],
                                  wsem.at[b]).wait()

        pltpu.sync_copy(idx_hbm.at[pl.ds(base, b_per_w)], idx_v)
        for b in range(NBUF):
            pltpu.sync_copy(cls_hbm, bufs.at[b, pl.ds(0, 1)])

        start_gather(0, 0)
        start_gather(1, 1)
        for i in range(2):
            wait_gather(i)
            start_write(i, i)
            start_gather(i + 2, i + 2)
        for i in range(2, 4):
            b = i % NBUF
            wait_gather(b)
            start_write(i, b)
            wait_write(i - 2, (b + 2) % NBUF)
            start_gather(i + 2, (b + 2) % NBUF)

        @pl.loop(1, b_per_w // NBUF - 1)
        def steady(g):
            for b in range(NBUF):
                i = g * NBUF + b
                wait_gather(b)
                start_write(i, b)
                wait_write(i - 2, (b + 2) % NBUF)
                start_gather(i + 2, (b + 2) % NBUF)

        for i in range(b_per_w - NBUF, b_per_w - 2):
            b = i % NBUF
            wait_gather(b)
            start_write(i, b)
            wait_write(i - 2, (b + 2) % NBUF)
            start_gather(i + 2, (b + 2) % NBUF)
        for i in range(b_per_w - 2, b_per_w):
            b = i % NBUF
            wait_gather(b)
            start_write(i, b)
            wait_write(i - 2, (b + 2) % NBUF)
        wait_write(b_per_w - 2, (b_per_w - 2) % NBUF)
        wait_write(b_per_w - 1, (b_per_w - 1) % NBUF)

    return emb_kernel(idx3, table, cls_row)
